# Initial kernel scaffold; baseline (speedup 1.0000x reference)
import jax
import jax.numpy as jnp
from jax.experimental import pallas as pl


def kernel(x, adj, W_e0, a_e0, W_e1, a_e1, W_d0, a_d0, W_d1, a_d1):
    raise NotImplementedError("write your pallas kernel here")


# speedup vs baseline: 1.0226x; 1.0226x over previous
import functools

import jax
import jax.numpy as jnp
from jax.experimental import pallas as pl

N = 4096
BM = 256


def _pack_kernel(adj_ref, m_ref):
    m_ref[:] = (adj_ref[:] > 0).astype(jnp.int8)


def _proj_kernel(h_ref, w_ref, a1_ref, a2_ref, wh_ref, f1_ref, f2_ref):
    wh = jnp.dot(h_ref[:], w_ref[:], preferred_element_type=jnp.float32)
    wh_ref[:] = wh
    f1_ref[:] = jnp.dot(wh, a1_ref[:], preferred_element_type=jnp.float32)
    f2_ref[:] = jnp.dot(wh, a2_ref[:], preferred_element_type=jnp.float32)


def _att_kernel(f1_ref, f2t_ref, mask_ref, wh_ref, out_ref):
    e = f1_ref[:] + f2t_ref[:]
    e = jnp.where(e > 0, e, 0.2 * e)
    e = jnp.where(mask_ref[:] != 0, e, jnp.float32(-9e15))
    m = jnp.max(e, axis=1, keepdims=True)
    p = jnp.exp(e - m)
    s = jnp.sum(p, axis=1, keepdims=True)
    acc = jnp.dot(p, wh_ref[:], preferred_element_type=jnp.float32)
    acc = acc / s
    out_ref[:] = jnp.where(acc > 0, acc, jnp.exp(jnp.minimum(acc, 0.0)) - 1.0)


def _gat_layer(h, mask8, W, a):
    din, d = W.shape
    grid = (N // BM,)
    wh, f1, f2 = pl.pallas_call(
        _proj_kernel,
        grid=grid,
        in_specs=[
            pl.BlockSpec((BM, din), lambda i: (i, 0)),
            pl.BlockSpec((din, d), lambda i: (0, 0)),
            pl.BlockSpec((d, 1), lambda i: (0, 0)),
            pl.BlockSpec((d, 1), lambda i: (0, 0)),
        ],
        out_specs=[
            pl.BlockSpec((BM, d), lambda i: (i, 0)),
            pl.BlockSpec((BM, 1), lambda i: (i, 0)),
            pl.BlockSpec((BM, 1), lambda i: (i, 0)),
        ],
        out_shape=[
            jax.ShapeDtypeStruct((N, d), jnp.float32),
            jax.ShapeDtypeStruct((N, 1), jnp.float32),
            jax.ShapeDtypeStruct((N, 1), jnp.float32),
        ],
    )(h, W, a[:d].reshape(d, 1), a[d:].reshape(d, 1))

    out = pl.pallas_call(
        _att_kernel,
        grid=grid,
        in_specs=[
            pl.BlockSpec((BM, 1), lambda i: (i, 0)),
            pl.BlockSpec((1, N), lambda i: (0, 0)),
            pl.BlockSpec((BM, N), lambda i: (i, 0)),
            pl.BlockSpec((N, d), lambda i: (0, 0)),
        ],
        out_specs=pl.BlockSpec((BM, d), lambda i: (i, 0)),
        out_shape=jax.ShapeDtypeStruct((N, d), jnp.float32),
    )(f1, f2.T, mask8, wh)
    return out


@jax.jit
def _run(x, adj, W_e0, a_e0, W_e1, a_e1, W_d0, a_d0, W_d1, a_d1):
    mask8 = pl.pallas_call(
        _pack_kernel,
        grid=(N // BM,),
        in_specs=[pl.BlockSpec((BM, N), lambda i: (i, 0))],
        out_specs=pl.BlockSpec((BM, N), lambda i: (i, 0)),
        out_shape=jax.ShapeDtypeStruct((N, N), jnp.int8),
    )(adj)

    h = x
    for W, a in ((W_e0, a_e0), (W_e1, a_e1)):
        h = _gat_layer(h, mask8, W, a)
    z = h
    for W, a in ((W_d0, a_d0), (W_d1, a_d1)):
        h = _gat_layer(h, mask8, W, a)
    return z, h


def kernel(x, adj, W_e0, a_e0, W_e1, a_e1, W_d0, a_d0, W_d1, a_d1):
    return _run(x, adj, W_e0, a_e0, W_e1, a_e1, W_d0, a_d0, W_d1, a_d1)


# speedup vs baseline: 1.1661x; 1.1403x over previous
import jax
import jax.numpy as jnp
from jax.experimental import pallas as pl

N = 4096
BM = 256


def _ext_width(d):
    return 128 * ((d + 1 + 127) // 128)


def _pack_kernel(adj_ref, m_ref):
    m_ref[:] = (adj_ref[:] > 0).astype(jnp.int8)


def _proj_kernel(h_ref, w_ref, a1_ref, a2_ref, whe_ref, f1_ref, f2_ref):
    d = w_ref.shape[1]
    wh = jnp.dot(h_ref[:], w_ref[:], preferred_element_type=jnp.float32)
    whe_ref[:, :d] = wh.astype(jnp.bfloat16)
    whe_ref[:, d:] = jnp.ones_like(whe_ref[:, d:])
    f1_ref[:] = jnp.dot(wh, a1_ref[:], preferred_element_type=jnp.float32)
    f2_ref[:] = jnp.dot(wh, a2_ref[:], preferred_element_type=jnp.float32)


def _att_kernel(f1_ref, f2t_ref, mask_ref, whe_ref, out_ref, *, d):
    f2 = f2t_ref[:]
    m2 = jnp.max(f2, axis=1, keepdims=True)
    fm = f1_ref[:] + m2
    m = jnp.maximum(fm, 0.2 * fm)
    e = f1_ref[:] + f2
    l = jnp.maximum(e, 0.2 * e)
    p = jnp.where(mask_ref[:] != 0, jnp.exp(l - m), 0.0)
    acc = jnp.dot(p.astype(jnp.bfloat16), whe_ref[:],
                  preferred_element_type=jnp.float32)
    s = acc[:, d:d + 1]
    r = acc[:, :d] / jnp.where(s > 0.0, s, 1.0)
    ones = jnp.ones((8, N), jnp.bfloat16)
    cm = jnp.dot(ones, whe_ref[:], preferred_element_type=jnp.float32)
    r = jnp.where(s > 0.0, r, cm[0:1, :d] * (1.0 / N))
    out_ref[:] = jnp.where(r > 0, r, jnp.exp(jnp.minimum(r, 0.0)) - 1.0)


def _gat_layer(h, mask8, W, a):
    din, d = W.shape
    ew = _ext_width(d)
    grid = (N // BM,)
    whe, f1, f2 = pl.pallas_call(
        _proj_kernel,
        grid=grid,
        in_specs=[
            pl.BlockSpec((BM, din), lambda i: (i, 0)),
            pl.BlockSpec((din, d), lambda i: (0, 0)),
            pl.BlockSpec((d, 1), lambda i: (0, 0)),
            pl.BlockSpec((d, 1), lambda i: (0, 0)),
        ],
        out_specs=[
            pl.BlockSpec((BM, ew), lambda i: (i, 0)),
            pl.BlockSpec((BM, 1), lambda i: (i, 0)),
            pl.BlockSpec((BM, 1), lambda i: (i, 0)),
        ],
        out_shape=[
            jax.ShapeDtypeStruct((N, ew), jnp.bfloat16),
            jax.ShapeDtypeStruct((N, 1), jnp.float32),
            jax.ShapeDtypeStruct((N, 1), jnp.float32),
        ],
    )(h, W, a[:d].reshape(d, 1), a[d:].reshape(d, 1))

    import functools
    out = pl.pallas_call(
        functools.partial(_att_kernel, d=d),
        grid=grid,
        in_specs=[
            pl.BlockSpec((BM, 1), lambda i: (i, 0)),
            pl.BlockSpec((1, N), lambda i: (0, 0)),
            pl.BlockSpec((BM, N), lambda i: (i, 0)),
            pl.BlockSpec((N, ew), lambda i: (0, 0)),
        ],
        out_specs=pl.BlockSpec((BM, d), lambda i: (i, 0)),
        out_shape=jax.ShapeDtypeStruct((N, d), jnp.float32),
    )(f1, f2.T, mask8, whe)
    return out


@jax.jit
def _run(x, adj, W_e0, a_e0, W_e1, a_e1, W_d0, a_d0, W_d1, a_d1):
    mask8 = pl.pallas_call(
        _pack_kernel,
        grid=(N // BM,),
        in_specs=[pl.BlockSpec((BM, N), lambda i: (i, 0))],
        out_specs=pl.BlockSpec((BM, N), lambda i: (i, 0)),
        out_shape=jax.ShapeDtypeStruct((N, N), jnp.int8),
    )(adj)

    h = x
    for W, a in ((W_e0, a_e0), (W_e1, a_e1)):
        h = _gat_layer(h, mask8, W, a)
    z = h
    for W, a in ((W_d0, a_d0), (W_d1, a_d1)):
        h = _gat_layer(h, mask8, W, a)
    return z, h


def kernel(x, adj, W_e0, a_e0, W_e1, a_e1, W_d0, a_d0, W_d1, a_d1):
    return _run(x, adj, W_e0, a_e0, W_e1, a_e1, W_d0, a_d0, W_d1, a_d1)


# speedup vs baseline: 1.3335x; 1.1436x over previous
import functools

import jax
import jax.numpy as jnp
from jax.experimental import pallas as pl

N = 4096
BM = 256


def _ext_width(d):
    return 128 * ((d + 1 + 127) // 128)


def _proj(h, W, a1, a2, whe_ref, f1_ref, f2_ref):
    d = W.shape[1]
    wh = jnp.dot(h, W, preferred_element_type=jnp.float32)
    whe_ref[:, :d] = wh.astype(jnp.bfloat16)
    whe_ref[:, d:] = jnp.ones_like(whe_ref[:, d:])
    f1_ref[:] = jnp.dot(wh, a1, preferred_element_type=jnp.float32)
    f2_ref[:] = jnp.dot(wh, a2, preferred_element_type=jnp.float32)


def _setup_kernel(adj_ref, x_ref, w_ref, a1_ref, a2_ref,
                  mask_ref, whe_ref, f1_ref, f2_ref):
    mask_ref[:] = (adj_ref[:] > 0).astype(jnp.int8)
    _proj(x_ref[:], w_ref[:], a1_ref[:], a2_ref[:], whe_ref, f1_ref, f2_ref)


def _att_body(f1_ref, f2t_ref, mask_ref, whe_ref, d):
    f2 = f2t_ref[:]
    m2 = jnp.max(f2, axis=1, keepdims=True)
    fm = f1_ref[:] + m2
    m = jnp.maximum(fm, 0.2 * fm)
    e = f1_ref[:] + f2
    l = jnp.maximum(e, 0.2 * e)
    p = jnp.where(mask_ref[:] != 0, jnp.exp(l - m), 0.0)
    acc = jnp.dot(p.astype(jnp.bfloat16), whe_ref[:],
                  preferred_element_type=jnp.float32)
    s = acc[:, d:d + 1]
    r = acc[:, :d] / jnp.where(s > 0.0, s, 1.0)
    ones = jnp.ones((8, N), jnp.bfloat16)
    cm = jnp.dot(ones, whe_ref[:], preferred_element_type=jnp.float32)
    r = jnp.where(s > 0.0, r, cm[0:1, :d] * (1.0 / N))
    return jnp.where(r > 0, r, jnp.exp(jnp.minimum(r, 0.0)) - 1.0)


def _att_next_kernel(f1_ref, f2t_ref, mask_ref, whe_ref,
                     w_ref, a1_ref, a2_ref, *out_refs, d, emit_h):
    h = _att_body(f1_ref, f2t_ref, mask_ref, whe_ref, d)
    if emit_h:
        h_ref, whe2_ref, f12_ref, f22_ref = out_refs
        h_ref[:] = h
    else:
        whe2_ref, f12_ref, f22_ref = out_refs
    _proj(h, w_ref[:], a1_ref[:], a2_ref[:], whe2_ref, f12_ref, f22_ref)


def _att_last_kernel(f1_ref, f2t_ref, mask_ref, whe_ref, out_ref, *, d):
    out_ref[:] = _att_body(f1_ref, f2t_ref, mask_ref, whe_ref, d)


def _att_next(f1, f2, mask8, whe, W, a, d, emit_h=False):
    dn = W.shape[1]
    ew = _ext_width(dn)
    grid = (N // BM,)
    ewp = whe.shape[1]
    in_specs = [
        pl.BlockSpec((BM, 1), lambda i: (i, 0)),
        pl.BlockSpec((1, N), lambda i: (0, 0)),
        pl.BlockSpec((BM, N), lambda i: (i, 0)),
        pl.BlockSpec((N, ewp), lambda i: (0, 0)),
        pl.BlockSpec((W.shape[0], dn), lambda i: (0, 0)),
        pl.BlockSpec((dn, 1), lambda i: (0, 0)),
        pl.BlockSpec((dn, 1), lambda i: (0, 0)),
    ]
    out_specs = [
        pl.BlockSpec((BM, ew), lambda i: (i, 0)),
        pl.BlockSpec((BM, 1), lambda i: (i, 0)),
        pl.BlockSpec((BM, 1), lambda i: (i, 0)),
    ]
    out_shape = [
        jax.ShapeDtypeStruct((N, ew), jnp.bfloat16),
        jax.ShapeDtypeStruct((N, 1), jnp.float32),
        jax.ShapeDtypeStruct((N, 1), jnp.float32),
    ]
    if emit_h:
        out_specs = [pl.BlockSpec((BM, d), lambda i: (i, 0))] + out_specs
        out_shape = [jax.ShapeDtypeStruct((N, d), jnp.float32)] + out_shape
    return pl.pallas_call(
        functools.partial(_att_next_kernel, d=d, emit_h=emit_h),
        grid=grid,
        in_specs=in_specs,
        out_specs=out_specs,
        out_shape=out_shape,
    )(f1, f2.T, mask8, whe, W, a[:dn].reshape(dn, 1), a[dn:].reshape(dn, 1))


@jax.jit
def _run(x, adj, W_e0, a_e0, W_e1, a_e1, W_d0, a_d0, W_d1, a_d1):
    grid = (N // BM,)
    d0 = W_e0.shape[1]
    ew0 = _ext_width(d0)
    mask8, whe, f1, f2 = pl.pallas_call(
        _setup_kernel,
        grid=grid,
        in_specs=[
            pl.BlockSpec((BM, N), lambda i: (i, 0)),
            pl.BlockSpec((BM, x.shape[1]), lambda i: (i, 0)),
            pl.BlockSpec((x.shape[1], d0), lambda i: (0, 0)),
            pl.BlockSpec((d0, 1), lambda i: (0, 0)),
            pl.BlockSpec((d0, 1), lambda i: (0, 0)),
        ],
        out_specs=[
            pl.BlockSpec((BM, N), lambda i: (i, 0)),
            pl.BlockSpec((BM, ew0), lambda i: (i, 0)),
            pl.BlockSpec((BM, 1), lambda i: (i, 0)),
            pl.BlockSpec((BM, 1), lambda i: (i, 0)),
        ],
        out_shape=[
            jax.ShapeDtypeStruct((N, N), jnp.int8),
            jax.ShapeDtypeStruct((N, ew0), jnp.bfloat16),
            jax.ShapeDtypeStruct((N, 1), jnp.float32),
            jax.ShapeDtypeStruct((N, 1), jnp.float32),
        ],
    )(adj, x, W_e0, a_e0[:d0].reshape(d0, 1), a_e0[d0:].reshape(d0, 1))

    whe, f1, f2 = _att_next(f1, f2, mask8, whe, W_e1, a_e1, d0)
    d1 = W_e1.shape[1]
    z, whe, f1, f2 = _att_next(f1, f2, mask8, whe, W_d0, a_d0, d1, emit_h=True)
    d2 = W_d0.shape[1]
    whe, f1, f2 = _att_next(f1, f2, mask8, whe, W_d1, a_d1, d2)
    d3 = W_d1.shape[1]
    x_hat = pl.pallas_call(
        functools.partial(_att_last_kernel, d=d3),
        grid=grid,
        in_specs=[
            pl.BlockSpec((BM, 1), lambda i: (i, 0)),
            pl.BlockSpec((1, N), lambda i: (0, 0)),
            pl.BlockSpec((BM, N), lambda i: (i, 0)),
            pl.BlockSpec((N, whe.shape[1]), lambda i: (0, 0)),
        ],
        out_specs=pl.BlockSpec((BM, d3), lambda i: (i, 0)),
        out_shape=jax.ShapeDtypeStruct((N, d3), jnp.float32),
    )(f1, f2.T, mask8, whe)
    return z, x_hat


def kernel(x, adj, W_e0, a_e0, W_e1, a_e1, W_d0, a_d0, W_d1, a_d1):
    return _run(x, adj, W_e0, a_e0, W_e1, a_e1, W_d0, a_d0, W_d1, a_d1)


# speedup vs baseline: 1.4251x; 1.0687x over previous
import functools

import jax
import jax.numpy as jnp
from jax.experimental import pallas as pl

N = 4096
BM = 256


def _ext_width(d):
    return 128 * ((d + 1 + 127) // 128)


def _proj(h, W, a1, a2, whe_ref, f1_ref, f2_ref):
    d = W.shape[1]
    wh = jnp.dot(h, W, preferred_element_type=jnp.float32)
    whe_ref[:, :d] = wh.astype(jnp.bfloat16)
    whe_ref[:, d:] = jnp.ones_like(whe_ref[:, d:])
    f1_ref[:] = jnp.dot(wh, a1, preferred_element_type=jnp.float32)
    f2_ref[:] = jnp.dot(wh, a2, preferred_element_type=jnp.float32)


def _setup_kernel(adj_ref, x_ref, w_ref, a1_ref, a2_ref,
                  mask_ref, whe_ref, f1_ref, f2_ref):
    mask_ref[:] = (adj_ref[:] > 0).astype(jnp.int8)
    _proj(x_ref[:], w_ref[:], a1_ref[:], a2_ref[:], whe_ref, f1_ref, f2_ref)


def _att_body(f1_ref, f2t_ref, mask_ref, whe_ref, d):
    log2e = jnp.float32(1.4426950408889634)
    f1 = f1_ref[:]
    f2 = f2t_ref[:]
    m2 = jnp.max(f2, axis=1, keepdims=True)
    fm = f1 + m2
    m = jnp.maximum(fm, 0.2 * fm)
    c1 = (f1 - m) * log2e
    c2 = (0.2 * f1 - m) * log2e
    g1 = f2 * log2e
    g2 = f2 * (0.2 * log2e)
    arg = jnp.maximum(c1 + g1, c2 + g2)
    p = jnp.where(mask_ref[:] != 0, jnp.exp2(arg), 0.0)
    acc = jnp.dot(p.astype(jnp.bfloat16), whe_ref[:],
                  preferred_element_type=jnp.float32)
    s = acc[:, d:d + 1]
    r = acc[:, :d] / jnp.where(s > 0.0, s, 1.0)
    ones = jnp.ones((8, N), jnp.bfloat16)
    cm = jnp.dot(ones, whe_ref[:], preferred_element_type=jnp.float32)
    r = jnp.where(s > 0.0, r, cm[0:1, :d] * (1.0 / N))
    return jnp.where(r > 0, r, jnp.exp(jnp.minimum(r, 0.0)) - 1.0)


def _att_next_kernel(f1_ref, f2t_ref, mask_ref, whe_ref,
                     w_ref, a1_ref, a2_ref, *out_refs, d, emit_h):
    h = _att_body(f1_ref, f2t_ref, mask_ref, whe_ref, d)
    if emit_h:
        h_ref, whe2_ref, f12_ref, f22_ref = out_refs
        h_ref[:] = h
    else:
        whe2_ref, f12_ref, f22_ref = out_refs
    _proj(h, w_ref[:], a1_ref[:], a2_ref[:], whe2_ref, f12_ref, f22_ref)


def _att_last_kernel(f1_ref, f2t_ref, mask_ref, whe_ref, out_ref, *, d):
    out_ref[:] = _att_body(f1_ref, f2t_ref, mask_ref, whe_ref, d)


def _att_next(f1, f2, mask8, whe, W, a, d, emit_h=False):
    dn = W.shape[1]
    ew = _ext_width(dn)
    grid = (N // BM,)
    ewp = whe.shape[1]
    in_specs = [
        pl.BlockSpec((BM, 1), lambda i: (i, 0)),
        pl.BlockSpec((1, N), lambda i: (0, 0)),
        pl.BlockSpec((BM, N), lambda i: (i, 0)),
        pl.BlockSpec((N, ewp), lambda i: (0, 0)),
        pl.BlockSpec((W.shape[0], dn), lambda i: (0, 0)),
        pl.BlockSpec((dn, 1), lambda i: (0, 0)),
        pl.BlockSpec((dn, 1), lambda i: (0, 0)),
    ]
    out_specs = [
        pl.BlockSpec((BM, ew), lambda i: (i, 0)),
        pl.BlockSpec((BM, 1), lambda i: (i, 0)),
        pl.BlockSpec((BM, 1), lambda i: (i, 0)),
    ]
    out_shape = [
        jax.ShapeDtypeStruct((N, ew), jnp.bfloat16),
        jax.ShapeDtypeStruct((N, 1), jnp.float32),
        jax.ShapeDtypeStruct((N, 1), jnp.float32),
    ]
    if emit_h:
        out_specs = [pl.BlockSpec((BM, d), lambda i: (i, 0))] + out_specs
        out_shape = [jax.ShapeDtypeStruct((N, d), jnp.float32)] + out_shape
    return pl.pallas_call(
        functools.partial(_att_next_kernel, d=d, emit_h=emit_h),
        grid=grid,
        in_specs=in_specs,
        out_specs=out_specs,
        out_shape=out_shape,
    )(f1, f2.T, mask8, whe, W, a[:dn].reshape(dn, 1), a[dn:].reshape(dn, 1))


@jax.jit
def _run(x, adj, W_e0, a_e0, W_e1, a_e1, W_d0, a_d0, W_d1, a_d1):
    grid = (N // BM,)
    d0 = W_e0.shape[1]
    ew0 = _ext_width(d0)
    mask8, whe, f1, f2 = pl.pallas_call(
        _setup_kernel,
        grid=grid,
        in_specs=[
            pl.BlockSpec((BM, N), lambda i: (i, 0)),
            pl.BlockSpec((BM, x.shape[1]), lambda i: (i, 0)),
            pl.BlockSpec((x.shape[1], d0), lambda i: (0, 0)),
            pl.BlockSpec((d0, 1), lambda i: (0, 0)),
            pl.BlockSpec((d0, 1), lambda i: (0, 0)),
        ],
        out_specs=[
            pl.BlockSpec((BM, N), lambda i: (i, 0)),
            pl.BlockSpec((BM, ew0), lambda i: (i, 0)),
            pl.BlockSpec((BM, 1), lambda i: (i, 0)),
            pl.BlockSpec((BM, 1), lambda i: (i, 0)),
        ],
        out_shape=[
            jax.ShapeDtypeStruct((N, N), jnp.int8),
            jax.ShapeDtypeStruct((N, ew0), jnp.bfloat16),
            jax.ShapeDtypeStruct((N, 1), jnp.float32),
            jax.ShapeDtypeStruct((N, 1), jnp.float32),
        ],
    )(adj, x, W_e0, a_e0[:d0].reshape(d0, 1), a_e0[d0:].reshape(d0, 1))

    whe, f1, f2 = _att_next(f1, f2, mask8, whe, W_e1, a_e1, d0)
    d1 = W_e1.shape[1]
    z, whe, f1, f2 = _att_next(f1, f2, mask8, whe, W_d0, a_d0, d1, emit_h=True)
    d2 = W_d0.shape[1]
    whe, f1, f2 = _att_next(f1, f2, mask8, whe, W_d1, a_d1, d2)
    d3 = W_d1.shape[1]
    x_hat = pl.pallas_call(
        functools.partial(_att_last_kernel, d=d3),
        grid=grid,
        in_specs=[
            pl.BlockSpec((BM, 1), lambda i: (i, 0)),
            pl.BlockSpec((1, N), lambda i: (0, 0)),
            pl.BlockSpec((BM, N), lambda i: (i, 0)),
            pl.BlockSpec((N, whe.shape[1]), lambda i: (0, 0)),
        ],
        out_specs=pl.BlockSpec((BM, d3), lambda i: (i, 0)),
        out_shape=jax.ShapeDtypeStruct((N, d3), jnp.float32),
    )(f1, f2.T, mask8, whe)
    return z, x_hat


def kernel(x, adj, W_e0, a_e0, W_e1, a_e1, W_d0, a_d0, W_d1, a_d1):
    return _run(x, adj, W_e0, a_e0, W_e1, a_e1, W_d0, a_d0, W_d1, a_d1)


# speedup vs baseline: 1.4808x; 1.0390x over previous
import functools

import jax
import jax.numpy as jnp
from jax.experimental import pallas as pl

N = 4096
BM = 256


def _ext_width(d):
    return 128 * ((d + 1 + 127) // 128)


def _proj(h, W, a1, a2, whe_ref, f1_ref, f2_ref):
    d = W.shape[1]
    wh = jnp.dot(h, W, preferred_element_type=jnp.float32)
    whe_ref[:, :d] = wh.astype(jnp.bfloat16)
    whe_ref[:, d:] = jnp.ones_like(whe_ref[:, d:])
    f1_ref[:] = jnp.dot(wh, a1, preferred_element_type=jnp.float32)
    f2_ref[:] = jnp.dot(wh, a2, preferred_element_type=jnp.float32)


def _setup_kernel(adj_ref, x_ref, w_ref, a1_ref, a2_ref,
                  mask_ref, whe_ref, f1_ref, f2_ref):
    mask_ref[:] = (adj_ref[:] > 0).astype(jnp.int8)
    _proj(x_ref[:], w_ref[:], a1_ref[:], a2_ref[:], whe_ref, f1_ref, f2_ref)


def _att_body(f1_ref, f2t_ref, mask_ref, whe_ref, d):
    log2e = jnp.float32(1.4426950408889634)
    f1 = f1_ref[:]
    f2 = f2t_ref[:]
    m2 = jnp.max(f2, axis=1, keepdims=True)
    fm = f1 + m2
    m = jnp.maximum(fm, 0.2 * fm)
    c1 = (f1 - m) * log2e
    c2 = (0.2 * f1 - m) * log2e
    g1 = f2 * log2e
    g2 = f2 * (0.2 * log2e)
    u1 = jnp.exp2(c1)
    u2 = jnp.exp2(c2)
    v1 = jnp.exp2(g1)
    v2 = jnp.exp2(g2)
    p = jnp.maximum(u1 * v1, u2 * v2)
    p = jnp.where(mask_ref[:] != 0, p, 0.0)
    acc = jnp.dot(p.astype(jnp.bfloat16), whe_ref[:],
                  preferred_element_type=jnp.float32)
    s = acc[:, d:d + 1]
    r = acc[:, :d] / jnp.where(s > 0.0, s, 1.0)
    ones = jnp.ones((8, N), jnp.bfloat16)
    cm = jnp.dot(ones, whe_ref[:], preferred_element_type=jnp.float32)
    r = jnp.where(s > 0.0, r, cm[0:1, :d] * (1.0 / N))
    return jnp.where(r > 0, r, jnp.exp(jnp.minimum(r, 0.0)) - 1.0)


def _att_next_kernel(f1_ref, f2t_ref, mask_ref, whe_ref,
                     w_ref, a1_ref, a2_ref, *out_refs, d, emit_h):
    h = _att_body(f1_ref, f2t_ref, mask_ref, whe_ref, d)
    if emit_h:
        h_ref, whe2_ref, f12_ref, f22_ref = out_refs
        h_ref[:] = h
    else:
        whe2_ref, f12_ref, f22_ref = out_refs
    _proj(h, w_ref[:], a1_ref[:], a2_ref[:], whe2_ref, f12_ref, f22_ref)


def _att_last_kernel(f1_ref, f2t_ref, mask_ref, whe_ref, out_ref, *, d):
    out_ref[:] = _att_body(f1_ref, f2t_ref, mask_ref, whe_ref, d)


def _att_next(f1, f2, mask8, whe, W, a, d, emit_h=False):
    dn = W.shape[1]
    ew = _ext_width(dn)
    grid = (N // BM,)
    ewp = whe.shape[1]
    in_specs = [
        pl.BlockSpec((BM, 1), lambda i: (i, 0)),
        pl.BlockSpec((1, N), lambda i: (0, 0)),
        pl.BlockSpec((BM, N), lambda i: (i, 0)),
        pl.BlockSpec((N, ewp), lambda i: (0, 0)),
        pl.BlockSpec((W.shape[0], dn), lambda i: (0, 0)),
        pl.BlockSpec((dn, 1), lambda i: (0, 0)),
        pl.BlockSpec((dn, 1), lambda i: (0, 0)),
    ]
    out_specs = [
        pl.BlockSpec((BM, ew), lambda i: (i, 0)),
        pl.BlockSpec((BM, 1), lambda i: (i, 0)),
        pl.BlockSpec((BM, 1), lambda i: (i, 0)),
    ]
    out_shape = [
        jax.ShapeDtypeStruct((N, ew), jnp.bfloat16),
        jax.ShapeDtypeStruct((N, 1), jnp.float32),
        jax.ShapeDtypeStruct((N, 1), jnp.float32),
    ]
    if emit_h:
        out_specs = [pl.BlockSpec((BM, d), lambda i: (i, 0))] + out_specs
        out_shape = [jax.ShapeDtypeStruct((N, d), jnp.float32)] + out_shape
    return pl.pallas_call(
        functools.partial(_att_next_kernel, d=d, emit_h=emit_h),
        grid=grid,
        in_specs=in_specs,
        out_specs=out_specs,
        out_shape=out_shape,
    )(f1, f2.T, mask8, whe, W, a[:dn].reshape(dn, 1), a[dn:].reshape(dn, 1))


@jax.jit
def _run(x, adj, W_e0, a_e0, W_e1, a_e1, W_d0, a_d0, W_d1, a_d1):
    grid = (N // BM,)
    d0 = W_e0.shape[1]
    ew0 = _ext_width(d0)
    mask8, whe, f1, f2 = pl.pallas_call(
        _setup_kernel,
        grid=grid,
        in_specs=[
            pl.BlockSpec((BM, N), lambda i: (i, 0)),
            pl.BlockSpec((BM, x.shape[1]), lambda i: (i, 0)),
            pl.BlockSpec((x.shape[1], d0), lambda i: (0, 0)),
            pl.BlockSpec((d0, 1), lambda i: (0, 0)),
            pl.BlockSpec((d0, 1), lambda i: (0, 0)),
        ],
        out_specs=[
            pl.BlockSpec((BM, N), lambda i: (i, 0)),
            pl.BlockSpec((BM, ew0), lambda i: (i, 0)),
            pl.BlockSpec((BM, 1), lambda i: (i, 0)),
            pl.BlockSpec((BM, 1), lambda i: (i, 0)),
        ],
        out_shape=[
            jax.ShapeDtypeStruct((N, N), jnp.int8),
            jax.ShapeDtypeStruct((N, ew0), jnp.bfloat16),
            jax.ShapeDtypeStruct((N, 1), jnp.float32),
            jax.ShapeDtypeStruct((N, 1), jnp.float32),
        ],
    )(adj, x, W_e0, a_e0[:d0].reshape(d0, 1), a_e0[d0:].reshape(d0, 1))

    whe, f1, f2 = _att_next(f1, f2, mask8, whe, W_e1, a_e1, d0)
    d1 = W_e1.shape[1]
    z, whe, f1, f2 = _att_next(f1, f2, mask8, whe, W_d0, a_d0, d1, emit_h=True)
    d2 = W_d0.shape[1]
    whe, f1, f2 = _att_next(f1, f2, mask8, whe, W_d1, a_d1, d2)
    d3 = W_d1.shape[1]
    x_hat = pl.pallas_call(
        functools.partial(_att_last_kernel, d=d3),
        grid=grid,
        in_specs=[
            pl.BlockSpec((BM, 1), lambda i: (i, 0)),
            pl.BlockSpec((1, N), lambda i: (0, 0)),
            pl.BlockSpec((BM, N), lambda i: (i, 0)),
            pl.BlockSpec((N, whe.shape[1]), lambda i: (0, 0)),
        ],
        out_specs=pl.BlockSpec((BM, d3), lambda i: (i, 0)),
        out_shape=jax.ShapeDtypeStruct((N, d3), jnp.float32),
    )(f1, f2.T, mask8, whe)
    return z, x_hat


def kernel(x, adj, W_e0, a_e0, W_e1, a_e1, W_d0, a_d0, W_d1, a_d1):
    return _run(x, adj, W_e0, a_e0, W_e1, a_e1, W_d0, a_d0, W_d1, a_d1)


# speedup vs baseline: 1.6547x; 1.1175x over previous
import functools

import jax
import jax.numpy as jnp
from jax.experimental import pallas as pl

N = 4096
BM = 256


def _ext_width(d):
    return 128 * ((d + 1 + 127) // 128)


def _proj(h, W, a1, a2, whe_ref, f1_ref, f2_ref):
    d = W.shape[1]
    wh = jnp.dot(h, W, preferred_element_type=jnp.float32)
    whe_ref[:, :d] = wh.astype(jnp.bfloat16)
    whe_ref[:, d:] = jnp.ones_like(whe_ref[:, d:])
    f1_ref[:] = jnp.dot(wh, a1, preferred_element_type=jnp.float32)
    f2_ref[:] = jnp.dot(wh, a2, preferred_element_type=jnp.float32)


def _setup_kernel(adj_ref, x_ref, w_ref, a1_ref, a2_ref,
                  mask_ref, whe_ref, f1_ref, f2_ref):
    mask_ref[:] = (adj_ref[:] > 0).astype(jnp.int8)
    _proj(x_ref[:], w_ref[:], a1_ref[:], a2_ref[:], whe_ref, f1_ref, f2_ref)


def _att_body(f1_ref, f2t_ref, mask_ref, whe_ref, d):
    log2e = jnp.float32(1.4426950408889634)
    f1 = f1_ref[:]
    f2 = f2t_ref[:]
    m2 = jnp.max(f2, axis=1, keepdims=True)
    fm = f1 + m2
    m = jnp.maximum(fm, 0.2 * fm)
    c1 = (f1 - m) * log2e
    c2 = (0.2 * f1 - m) * log2e
    g1 = f2 * log2e
    g2 = f2 * (0.2 * log2e)
    u1 = jnp.exp2(c1).astype(jnp.bfloat16)
    u2 = jnp.exp2(c2).astype(jnp.bfloat16)
    v1 = jnp.exp2(g1).astype(jnp.bfloat16)
    v2 = jnp.exp2(g2).astype(jnp.bfloat16)
    p = jnp.maximum(u1 * v1, u2 * v2)
    p = jnp.where(mask_ref[:] != 0, p, jnp.bfloat16(0.0))
    acc = jnp.dot(p, whe_ref[:],
                  preferred_element_type=jnp.float32)
    s = acc[:, d:d + 1]
    r = acc[:, :d] / jnp.where(s > 0.0, s, 1.0)
    ones = jnp.ones((8, N), jnp.bfloat16)
    cm = jnp.dot(ones, whe_ref[:], preferred_element_type=jnp.float32)
    r = jnp.where(s > 0.0, r, cm[0:1, :d] * (1.0 / N))
    return jnp.where(r > 0, r, jnp.exp(jnp.minimum(r, 0.0)) - 1.0)


def _att_next_kernel(f1_ref, f2t_ref, mask_ref, whe_ref,
                     w_ref, a1_ref, a2_ref, *out_refs, d, emit_h):
    h = _att_body(f1_ref, f2t_ref, mask_ref, whe_ref, d)
    if emit_h:
        h_ref, whe2_ref, f12_ref, f22_ref = out_refs
        h_ref[:] = h
    else:
        whe2_ref, f12_ref, f22_ref = out_refs
    _proj(h, w_ref[:], a1_ref[:], a2_ref[:], whe2_ref, f12_ref, f22_ref)


def _att_last_kernel(f1_ref, f2t_ref, mask_ref, whe_ref, out_ref, *, d):
    out_ref[:] = _att_body(f1_ref, f2t_ref, mask_ref, whe_ref, d)


def _att_next(f1, f2, mask8, whe, W, a, d, emit_h=False):
    dn = W.shape[1]
    ew = _ext_width(dn)
    grid = (N // BM,)
    ewp = whe.shape[1]
    in_specs = [
        pl.BlockSpec((BM, 1), lambda i: (i, 0)),
        pl.BlockSpec((1, N), lambda i: (0, 0)),
        pl.BlockSpec((BM, N), lambda i: (i, 0)),
        pl.BlockSpec((N, ewp), lambda i: (0, 0)),
        pl.BlockSpec((W.shape[0], dn), lambda i: (0, 0)),
        pl.BlockSpec((dn, 1), lambda i: (0, 0)),
        pl.BlockSpec((dn, 1), lambda i: (0, 0)),
    ]
    out_specs = [
        pl.BlockSpec((BM, ew), lambda i: (i, 0)),
        pl.BlockSpec((BM, 1), lambda i: (i, 0)),
        pl.BlockSpec((BM, 1), lambda i: (i, 0)),
    ]
    out_shape = [
        jax.ShapeDtypeStruct((N, ew), jnp.bfloat16),
        jax.ShapeDtypeStruct((N, 1), jnp.float32),
        jax.ShapeDtypeStruct((N, 1), jnp.float32),
    ]
    if emit_h:
        out_specs = [pl.BlockSpec((BM, d), lambda i: (i, 0))] + out_specs
        out_shape = [jax.ShapeDtypeStruct((N, d), jnp.float32)] + out_shape
    return pl.pallas_call(
        functools.partial(_att_next_kernel, d=d, emit_h=emit_h),
        grid=grid,
        in_specs=in_specs,
        out_specs=out_specs,
        out_shape=out_shape,
    )(f1, f2.T, mask8, whe, W, a[:dn].reshape(dn, 1), a[dn:].reshape(dn, 1))


@jax.jit
def _run(x, adj, W_e0, a_e0, W_e1, a_e1, W_d0, a_d0, W_d1, a_d1):
    grid = (N // BM,)
    d0 = W_e0.shape[1]
    ew0 = _ext_width(d0)
    mask8, whe, f1, f2 = pl.pallas_call(
        _setup_kernel,
        grid=grid,
        in_specs=[
            pl.BlockSpec((BM, N), lambda i: (i, 0)),
            pl.BlockSpec((BM, x.shape[1]), lambda i: (i, 0)),
            pl.BlockSpec((x.shape[1], d0), lambda i: (0, 0)),
            pl.BlockSpec((d0, 1), lambda i: (0, 0)),
            pl.BlockSpec((d0, 1), lambda i: (0, 0)),
        ],
        out_specs=[
            pl.BlockSpec((BM, N), lambda i: (i, 0)),
            pl.BlockSpec((BM, ew0), lambda i: (i, 0)),
            pl.BlockSpec((BM, 1), lambda i: (i, 0)),
            pl.BlockSpec((BM, 1), lambda i: (i, 0)),
        ],
        out_shape=[
            jax.ShapeDtypeStruct((N, N), jnp.int8),
            jax.ShapeDtypeStruct((N, ew0), jnp.bfloat16),
            jax.ShapeDtypeStruct((N, 1), jnp.float32),
            jax.ShapeDtypeStruct((N, 1), jnp.float32),
        ],
    )(adj, x, W_e0, a_e0[:d0].reshape(d0, 1), a_e0[d0:].reshape(d0, 1))

    whe, f1, f2 = _att_next(f1, f2, mask8, whe, W_e1, a_e1, d0)
    d1 = W_e1.shape[1]
    z, whe, f1, f2 = _att_next(f1, f2, mask8, whe, W_d0, a_d0, d1, emit_h=True)
    d2 = W_d0.shape[1]
    whe, f1, f2 = _att_next(f1, f2, mask8, whe, W_d1, a_d1, d2)
    d3 = W_d1.shape[1]
    x_hat = pl.pallas_call(
        functools.partial(_att_last_kernel, d=d3),
        grid=grid,
        in_specs=[
            pl.BlockSpec((BM, 1), lambda i: (i, 0)),
            pl.BlockSpec((1, N), lambda i: (0, 0)),
            pl.BlockSpec((BM, N), lambda i: (i, 0)),
            pl.BlockSpec((N, whe.shape[1]), lambda i: (0, 0)),
        ],
        out_specs=pl.BlockSpec((BM, d3), lambda i: (i, 0)),
        out_shape=jax.ShapeDtypeStruct((N, d3), jnp.float32),
    )(f1, f2.T, mask8, whe)
    return z, x_hat


def kernel(x, adj, W_e0, a_e0, W_e1, a_e1, W_d0, a_d0, W_d1, a_d1):
    return _run(x, adj, W_e0, a_e0, W_e1, a_e1, W_d0, a_d0, W_d1, a_d1)


# speedup vs baseline: 1.9822x; 1.1979x over previous
import functools

import jax
import jax.numpy as jnp
from jax.experimental import pallas as pl

N = 4096
BM = 512


def _ext_width(d):
    return 128 * ((d + 1 + 127) // 128)


def _proj(h, W, a1, a2, whe_ref, f1_ref, f2_ref):
    d = W.shape[1]
    wh = jnp.dot(h, W, preferred_element_type=jnp.float32)
    whe_ref[:, :d] = wh.astype(jnp.bfloat16)
    whe_ref[:, d:] = jnp.ones_like(whe_ref[:, d:])
    f1_ref[:] = jnp.dot(wh, a1, preferred_element_type=jnp.float32)
    f2_ref[:] = jnp.dot(wh, a2, preferred_element_type=jnp.float32)


def _setup_kernel(adj_ref, x_ref, w_ref, a1_ref, a2_ref,
                  mask_ref, whe_ref, f1_ref, f2_ref):
    mask_ref[:] = (adj_ref[:] > 0).astype(jnp.int8)
    _proj(x_ref[:], w_ref[:], a1_ref[:], a2_ref[:], whe_ref, f1_ref, f2_ref)


def _att_body(f1_ref, f2t_ref, mask_ref, whe_ref, d):
    log2e = jnp.float32(1.4426950408889634)
    f1 = f1_ref[:]
    f2 = f2t_ref[:]
    m2 = jnp.max(f2, axis=1, keepdims=True)
    fm = f1 + m2
    m = jnp.maximum(fm, 0.2 * fm)
    c1 = (f1 - m) * log2e
    c2 = (0.2 * f1 - m) * log2e
    g1 = f2 * log2e
    g2 = f2 * (0.2 * log2e)
    u1 = jnp.exp2(c1).astype(jnp.bfloat16)
    u2 = jnp.exp2(c2).astype(jnp.bfloat16)
    v1 = jnp.exp2(g1).astype(jnp.bfloat16)
    v2 = jnp.exp2(g2).astype(jnp.bfloat16)
    p = jnp.maximum(u1 * v1, u2 * v2)
    p = jnp.where(mask_ref[:] != 0, p, jnp.bfloat16(0.0))
    acc = jnp.dot(p, whe_ref[:],
                  preferred_element_type=jnp.float32)
    s = acc[:, d:d + 1]
    r = acc[:, :d] / jnp.where(s > 0.0, s, 1.0)
    ones = jnp.ones((8, N), jnp.bfloat16)
    cm = jnp.dot(ones, whe_ref[:], preferred_element_type=jnp.float32)
    r = jnp.where(s > 0.0, r, cm[0:1, :d] * (1.0 / N))
    return jnp.where(r > 0, r, jnp.exp(jnp.minimum(r, 0.0)) - 1.0)


def _att_next_kernel(f1_ref, f2t_ref, mask_ref, whe_ref,
                     w_ref, a1_ref, a2_ref, *out_refs, d, emit_h):
    h = _att_body(f1_ref, f2t_ref, mask_ref, whe_ref, d)
    if emit_h:
        h_ref, whe2_ref, f12_ref, f22_ref = out_refs
        h_ref[:] = h
    else:
        whe2_ref, f12_ref, f22_ref = out_refs
    _proj(h, w_ref[:], a1_ref[:], a2_ref[:], whe2_ref, f12_ref, f22_ref)


def _att_last_kernel(f1_ref, f2t_ref, mask_ref, whe_ref, out_ref, *, d):
    out_ref[:] = _att_body(f1_ref, f2t_ref, mask_ref, whe_ref, d)


def _att_next(f1, f2, mask8, whe, W, a, d, emit_h=False):
    dn = W.shape[1]
    ew = _ext_width(dn)
    grid = (N // BM,)
    ewp = whe.shape[1]
    in_specs = [
        pl.BlockSpec((BM, 1), lambda i: (i, 0)),
        pl.BlockSpec((1, N), lambda i: (0, 0)),
        pl.BlockSpec((BM, N), lambda i: (i, 0)),
        pl.BlockSpec((N, ewp), lambda i: (0, 0)),
        pl.BlockSpec((W.shape[0], dn), lambda i: (0, 0)),
        pl.BlockSpec((dn, 1), lambda i: (0, 0)),
        pl.BlockSpec((dn, 1), lambda i: (0, 0)),
    ]
    out_specs = [
        pl.BlockSpec((BM, ew), lambda i: (i, 0)),
        pl.BlockSpec((BM, 1), lambda i: (i, 0)),
        pl.BlockSpec((BM, 1), lambda i: (i, 0)),
    ]
    out_shape = [
        jax.ShapeDtypeStruct((N, ew), jnp.bfloat16),
        jax.ShapeDtypeStruct((N, 1), jnp.float32),
        jax.ShapeDtypeStruct((N, 1), jnp.float32),
    ]
    if emit_h:
        out_specs = [pl.BlockSpec((BM, d), lambda i: (i, 0))] + out_specs
        out_shape = [jax.ShapeDtypeStruct((N, d), jnp.float32)] + out_shape
    return pl.pallas_call(
        functools.partial(_att_next_kernel, d=d, emit_h=emit_h),
        grid=grid,
        in_specs=in_specs,
        out_specs=out_specs,
        out_shape=out_shape,
    )(f1, f2.T, mask8, whe, W, a[:dn].reshape(dn, 1), a[dn:].reshape(dn, 1))


@jax.jit
def _run(x, adj, W_e0, a_e0, W_e1, a_e1, W_d0, a_d0, W_d1, a_d1):
    grid = (N // BM,)
    d0 = W_e0.shape[1]
    ew0 = _ext_width(d0)
    mask8, whe, f1, f2 = pl.pallas_call(
        _setup_kernel,
        grid=grid,
        in_specs=[
            pl.BlockSpec((BM, N), lambda i: (i, 0)),
            pl.BlockSpec((BM, x.shape[1]), lambda i: (i, 0)),
            pl.BlockSpec((x.shape[1], d0), lambda i: (0, 0)),
            pl.BlockSpec((d0, 1), lambda i: (0, 0)),
            pl.BlockSpec((d0, 1), lambda i: (0, 0)),
        ],
        out_specs=[
            pl.BlockSpec((BM, N), lambda i: (i, 0)),
            pl.BlockSpec((BM, ew0), lambda i: (i, 0)),
            pl.BlockSpec((BM, 1), lambda i: (i, 0)),
            pl.BlockSpec((BM, 1), lambda i: (i, 0)),
        ],
        out_shape=[
            jax.ShapeDtypeStruct((N, N), jnp.int8),
            jax.ShapeDtypeStruct((N, ew0), jnp.bfloat16),
            jax.ShapeDtypeStruct((N, 1), jnp.float32),
            jax.ShapeDtypeStruct((N, 1), jnp.float32),
        ],
    )(adj, x, W_e0, a_e0[:d0].reshape(d0, 1), a_e0[d0:].reshape(d0, 1))

    whe, f1, f2 = _att_next(f1, f2, mask8, whe, W_e1, a_e1, d0)
    d1 = W_e1.shape[1]
    z, whe, f1, f2 = _att_next(f1, f2, mask8, whe, W_d0, a_d0, d1, emit_h=True)
    d2 = W_d0.shape[1]
    whe, f1, f2 = _att_next(f1, f2, mask8, whe, W_d1, a_d1, d2)
    d3 = W_d1.shape[1]
    x_hat = pl.pallas_call(
        functools.partial(_att_last_kernel, d=d3),
        grid=grid,
        in_specs=[
            pl.BlockSpec((BM, 1), lambda i: (i, 0)),
            pl.BlockSpec((1, N), lambda i: (0, 0)),
            pl.BlockSpec((BM, N), lambda i: (i, 0)),
            pl.BlockSpec((N, whe.shape[1]), lambda i: (0, 0)),
        ],
        out_specs=pl.BlockSpec((BM, d3), lambda i: (i, 0)),
        out_shape=jax.ShapeDtypeStruct((N, d3), jnp.float32),
    )(f1, f2.T, mask8, whe)
    return z, x_hat


def kernel(x, adj, W_e0, a_e0, W_e1, a_e1, W_d0, a_d0, W_d1, a_d1):
    return _run(x, adj, W_e0, a_e0, W_e1, a_e1, W_d0, a_d0, W_d1, a_d1)


# speedup vs baseline: 2.0648x; 1.0417x over previous
import functools

import jax
import jax.numpy as jnp
from jax.experimental import pallas as pl

N = 4096
BM = 1024


def _ext_width(d):
    return 128 * ((d + 1 + 127) // 128)


def _proj(h, W, a1, a2, whe_ref, f1_ref, f2_ref):
    d = W.shape[1]
    wh = jnp.dot(h, W, preferred_element_type=jnp.float32)
    whe_ref[:, :d] = wh.astype(jnp.bfloat16)
    whe_ref[:, d:] = jnp.ones_like(whe_ref[:, d:])
    f1_ref[:] = jnp.dot(wh, a1, preferred_element_type=jnp.float32)
    f2_ref[:] = jnp.dot(wh, a2, preferred_element_type=jnp.float32)


def _setup_kernel(adj_ref, x_ref, w_ref, a1_ref, a2_ref,
                  mask_ref, whe_ref, f1_ref, f2_ref):
    mask_ref[:] = (adj_ref[:] > 0).astype(jnp.int8)
    _proj(x_ref[:], w_ref[:], a1_ref[:], a2_ref[:], whe_ref, f1_ref, f2_ref)


def _att_body(f1_ref, f2t_ref, mask_ref, whe_ref, d):
    log2e = jnp.float32(1.4426950408889634)
    f1 = f1_ref[:]
    f2 = f2t_ref[:]
    m2 = jnp.max(f2, axis=1, keepdims=True)
    fm = f1 + m2
    m = jnp.maximum(fm, 0.2 * fm)
    c1 = (f1 - m) * log2e
    c2 = (0.2 * f1 - m) * log2e
    g1 = f2 * log2e
    g2 = f2 * (0.2 * log2e)
    u1 = jnp.exp2(c1).astype(jnp.bfloat16)
    u2 = jnp.exp2(c2).astype(jnp.bfloat16)
    v1 = jnp.exp2(g1).astype(jnp.bfloat16)
    v2 = jnp.exp2(g2).astype(jnp.bfloat16)
    p = jnp.maximum(u1 * v1, u2 * v2)
    p = jnp.where(mask_ref[:] != 0, p, jnp.bfloat16(0.0))
    acc = jnp.dot(p, whe_ref[:],
                  preferred_element_type=jnp.float32)
    s = acc[:, d:d + 1]
    r = acc[:, :d] / jnp.where(s > 0.0, s, 1.0)
    ones = jnp.ones((8, N), jnp.bfloat16)
    cm = jnp.dot(ones, whe_ref[:], preferred_element_type=jnp.float32)
    r = jnp.where(s > 0.0, r, cm[0:1, :d] * (1.0 / N))
    return jnp.where(r > 0, r, jnp.exp(jnp.minimum(r, 0.0)) - 1.0)


def _att_next_kernel(f1_ref, f2t_ref, mask_ref, whe_ref,
                     w_ref, a1_ref, a2_ref, *out_refs, d, emit_h):
    h = _att_body(f1_ref, f2t_ref, mask_ref, whe_ref, d)
    if emit_h:
        h_ref, whe2_ref, f12_ref, f22_ref = out_refs
        h_ref[:] = h
    else:
        whe2_ref, f12_ref, f22_ref = out_refs
    _proj(h, w_ref[:], a1_ref[:], a2_ref[:], whe2_ref, f12_ref, f22_ref)


def _att_last_kernel(f1_ref, f2t_ref, mask_ref, whe_ref, out_ref, *, d):
    out_ref[:] = _att_body(f1_ref, f2t_ref, mask_ref, whe_ref, d)


def _att_next(f1, f2, mask8, whe, W, a, d, emit_h=False):
    dn = W.shape[1]
    ew = _ext_width(dn)
    grid = (N // BM,)
    ewp = whe.shape[1]
    in_specs = [
        pl.BlockSpec((BM, 1), lambda i: (i, 0)),
        pl.BlockSpec((1, N), lambda i: (0, 0)),
        pl.BlockSpec((BM, N), lambda i: (i, 0)),
        pl.BlockSpec((N, ewp), lambda i: (0, 0)),
        pl.BlockSpec((W.shape[0], dn), lambda i: (0, 0)),
        pl.BlockSpec((dn, 1), lambda i: (0, 0)),
        pl.BlockSpec((dn, 1), lambda i: (0, 0)),
    ]
    out_specs = [
        pl.BlockSpec((BM, ew), lambda i: (i, 0)),
        pl.BlockSpec((BM, 1), lambda i: (i, 0)),
        pl.BlockSpec((BM, 1), lambda i: (i, 0)),
    ]
    out_shape = [
        jax.ShapeDtypeStruct((N, ew), jnp.bfloat16),
        jax.ShapeDtypeStruct((N, 1), jnp.float32),
        jax.ShapeDtypeStruct((N, 1), jnp.float32),
    ]
    if emit_h:
        out_specs = [pl.BlockSpec((BM, d), lambda i: (i, 0))] + out_specs
        out_shape = [jax.ShapeDtypeStruct((N, d), jnp.float32)] + out_shape
    return pl.pallas_call(
        functools.partial(_att_next_kernel, d=d, emit_h=emit_h),
        grid=grid,
        in_specs=in_specs,
        out_specs=out_specs,
        out_shape=out_shape,
    )(f1, f2.T, mask8, whe, W, a[:dn].reshape(dn, 1), a[dn:].reshape(dn, 1))


@jax.jit
def _run(x, adj, W_e0, a_e0, W_e1, a_e1, W_d0, a_d0, W_d1, a_d1):
    grid = (N // BM,)
    d0 = W_e0.shape[1]
    ew0 = _ext_width(d0)
    mask8, whe, f1, f2 = pl.pallas_call(
        _setup_kernel,
        grid=grid,
        in_specs=[
            pl.BlockSpec((BM, N), lambda i: (i, 0)),
            pl.BlockSpec((BM, x.shape[1]), lambda i: (i, 0)),
            pl.BlockSpec((x.shape[1], d0), lambda i: (0, 0)),
            pl.BlockSpec((d0, 1), lambda i: (0, 0)),
            pl.BlockSpec((d0, 1), lambda i: (0, 0)),
        ],
        out_specs=[
            pl.BlockSpec((BM, N), lambda i: (i, 0)),
            pl.BlockSpec((BM, ew0), lambda i: (i, 0)),
            pl.BlockSpec((BM, 1), lambda i: (i, 0)),
            pl.BlockSpec((BM, 1), lambda i: (i, 0)),
        ],
        out_shape=[
            jax.ShapeDtypeStruct((N, N), jnp.int8),
            jax.ShapeDtypeStruct((N, ew0), jnp.bfloat16),
            jax.ShapeDtypeStruct((N, 1), jnp.float32),
            jax.ShapeDtypeStruct((N, 1), jnp.float32),
        ],
    )(adj, x, W_e0, a_e0[:d0].reshape(d0, 1), a_e0[d0:].reshape(d0, 1))

    whe, f1, f2 = _att_next(f1, f2, mask8, whe, W_e1, a_e1, d0)
    d1 = W_e1.shape[1]
    z, whe, f1, f2 = _att_next(f1, f2, mask8, whe, W_d0, a_d0, d1, emit_h=True)
    d2 = W_d0.shape[1]
    whe, f1, f2 = _att_next(f1, f2, mask8, whe, W_d1, a_d1, d2)
    d3 = W_d1.shape[1]
    x_hat = pl.pallas_call(
        functools.partial(_att_last_kernel, d=d3),
        grid=grid,
        in_specs=[
            pl.BlockSpec((BM, 1), lambda i: (i, 0)),
            pl.BlockSpec((1, N), lambda i: (0, 0)),
            pl.BlockSpec((BM, N), lambda i: (i, 0)),
            pl.BlockSpec((N, whe.shape[1]), lambda i: (0, 0)),
        ],
        out_specs=pl.BlockSpec((BM, d3), lambda i: (i, 0)),
        out_shape=jax.ShapeDtypeStruct((N, d3), jnp.float32),
    )(f1, f2.T, mask8, whe)
    return z, x_hat


def kernel(x, adj, W_e0, a_e0, W_e1, a_e1, W_d0, a_d0, W_d1, a_d1):
    return _run(x, adj, W_e0, a_e0, W_e1, a_e1, W_d0, a_d0, W_d1, a_d1)
